# Initial kernel scaffold; baseline (speedup 1.0000x reference)
import jax
import jax.numpy as jnp
from jax.experimental import pallas as pl


def kernel(x, edge_index, W_emb1, b_emb1, Ws1, Wn1, Ws2, Wn2, W_emb2, b_emb2):
    raise NotImplementedError("write your pallas kernel here")


# speedup vs baseline: 4.3462x; 4.3462x over previous
import functools

import jax
import jax.numpy as jnp
from jax import lax
from jax.experimental import pallas as pl
from jax.experimental.pallas import tpu as pltpu
from jax.experimental.pallas import tpu_sc as plsc

N = 10000
H = 128
NUM_WORKERS = 32
CHUNK = 128
ROW_BLK = 1000
AGG_ROWS = 10240
ZERO_BLK = 128


def _gelu(v):
    return 0.5 * v * (1.0 + lax.erf(v * 0.7071067811865475))


def _mm(a, b_t):
    return lax.dot_general(a, b_t, (((1,), (1,)), ((), ())),
                           preferred_element_type=jnp.float32)


def _emb1_body(x_ref, w_ref, b_ref, o_ref):
    o_ref[...] = _gelu(_mm(x_ref[...], w_ref[...]) + b_ref[...])


def _emb1(x, w, b):
    n = x.shape[0]
    grid = (n // ROW_BLK,)
    return pl.pallas_call(
        _emb1_body,
        grid=grid,
        in_specs=[
            pl.BlockSpec((ROW_BLK, H), lambda i: (i, 0)),
            pl.BlockSpec((H, H), lambda i: (0, 0)),
            pl.BlockSpec((1, H), lambda i: (0, 0)),
        ],
        out_specs=pl.BlockSpec((ROW_BLK, H), lambda i: (i, 0)),
        out_shape=jax.ShapeDtypeStruct((n, H), jnp.float32),
    )(x, w, b.reshape(1, H))


def _layer_body(h_ref, p0_ref, p1_ref, ws_ref, wn_ref, o_ref):
    h = h_ref[...]
    agg = p0_ref[...] + p1_ref[...]
    o_ref[...] = _gelu(_mm(h, ws_ref[...]) + _mm(agg, wn_ref[...])) + h


def _layer(h, p0, p1, ws, wn):
    n = h.shape[0]
    grid = (n // ROW_BLK,)
    blk = pl.BlockSpec((ROW_BLK, H), lambda i: (i, 0))
    wblk = pl.BlockSpec((H, H), lambda i: (0, 0))
    return pl.pallas_call(
        _layer_body,
        grid=grid,
        in_specs=[blk, blk, blk, wblk, wblk],
        out_specs=blk,
        out_shape=jax.ShapeDtypeStruct((n, H), jnp.float32),
    )(h, p0, p1, ws, wn)


def _final_body(h_ref, p0_ref, p1_ref, ws_ref, wn_ref, w2_ref, b2_ref, o_ref):
    h = h_ref[...]
    agg = p0_ref[...] + p1_ref[...]
    h2 = _gelu(_mm(h, ws_ref[...]) + _mm(agg, wn_ref[...])) + h
    proj = jnp.sum(h2 * w2_ref[...], axis=1, keepdims=True) + b2_ref[0, 0]
    o_ref[...] = jnp.tanh(proj)


def _final(h, p0, p1, ws, wn, w2, b2):
    n = h.shape[0]
    grid = (n // ROW_BLK,)
    blk = pl.BlockSpec((ROW_BLK, H), lambda i: (i, 0))
    wblk = pl.BlockSpec((H, H), lambda i: (0, 0))
    return pl.pallas_call(
        _final_body,
        grid=grid,
        in_specs=[blk, blk, blk, wblk, wblk,
                  pl.BlockSpec((1, H), lambda i: (0, 0)),
                  pl.BlockSpec((1, 1), lambda i: (0, 0))],
        out_specs=pl.BlockSpec((ROW_BLK, 1), lambda i: (i, 0)),
        out_shape=jax.ShapeDtypeStruct((n, 1), jnp.float32),
    )(h, p0, p1, ws, wn, w2, b2.reshape(1, 1))


def _make_seg_sum(steps_per_worker):
    rows_per_tile = AGG_ROWS // 16
    mesh = plsc.VectorSubcoreMesh(core_axis_name="c", subcore_axis_name="s")

    @functools.partial(
        pl.kernel,
        mesh=mesh,
        out_type=jax.ShapeDtypeStruct((2 * AGG_ROWS, H), jnp.float32),
        scratch_types=[
            pltpu.VMEM((steps_per_worker, CHUNK), jnp.int32),
            pltpu.VMEM((steps_per_worker, CHUNK), jnp.int32),
            pltpu.VMEM((CHUNK, H), jnp.float32),
            pltpu.VMEM_SHARED((AGG_ROWS, H), jnp.float32),
            pltpu.SemaphoreType.DMA,
        ],
    )
    def seg_sum(h_hbm, src_hbm, dst_hbm, zeros_hbm, out_hbm,
                src_v, dst_v, rows_v, agg_sh, sem):
        c = lax.axis_index("c")
        s = lax.axis_index("s")
        wid = c * 16 + s

        pltpu.sync_copy(zeros_hbm, rows_v)
        for z in range(rows_per_tile // ZERO_BLK):
            pltpu.sync_copy(
                rows_v.at[pl.ds(0, ZERO_BLK)],
                agg_sh.at[pl.ds(s * rows_per_tile + z * ZERO_BLK, ZERO_BLK)])

        pltpu.sync_copy(src_hbm.at[wid], src_v)
        pltpu.sync_copy(dst_hbm.at[wid], dst_v)
        plsc.subcore_barrier()

        def body(i, carry):
            pltpu.async_copy(h_hbm.at[src_v.at[i]], rows_v, sem).wait()
            pltpu.sync_copy(rows_v, agg_sh.at[dst_v.at[i]], add=True)
            return carry

        lax.fori_loop(0, steps_per_worker, body, 0)
        plsc.subcore_barrier()

        for z in range(rows_per_tile // ZERO_BLK):
            off = s * rows_per_tile + z * ZERO_BLK
            pltpu.sync_copy(
                agg_sh.at[pl.ds(off, ZERO_BLK)],
                out_hbm.at[pl.ds(c * AGG_ROWS + off, ZERO_BLK)])

    return seg_sum


def kernel(x, edge_index, W_emb1, b_emb1, Ws1, Wn1, Ws2, Wn2, W_emb2, b_emb2):
    e = edge_index.shape[1]
    pad = (-e) % (NUM_WORKERS * CHUNK)
    src = jnp.concatenate(
        [edge_index[0], jnp.zeros((pad,), jnp.int32)])
    dst = jnp.concatenate(
        [edge_index[1], jnp.full((pad,), AGG_ROWS - 1, jnp.int32)])
    steps = (e + pad) // (NUM_WORKERS * CHUNK)
    src3 = src.reshape(NUM_WORKERS, steps, CHUNK)
    dst3 = dst.reshape(NUM_WORKERS, steps, CHUNK)
    zeros = jnp.zeros((ZERO_BLK, H), jnp.float32)

    seg_sum = _make_seg_sum(steps)

    h0 = _emb1(x, W_emb1, b_emb1)
    parts1 = seg_sum(h0, src3, dst3, zeros)
    h1 = _layer(h0, parts1[:N], parts1[AGG_ROWS:AGG_ROWS + N], Ws1, Wn1)
    parts2 = seg_sum(h1, src3, dst3, zeros)
    return _final(h1, parts2[:N], parts2[AGG_ROWS:AGG_ROWS + N],
                  Ws2, Wn2, W_emb2, b_emb2)
